# baseline (speedup 1.0000x reference)
import functools

import numpy as np
import jax
import jax.numpy as jnp
from jax import lax
from jax.experimental import pallas as pl
from jax.experimental.pallas import tpu as pltpu
from jax.experimental.pallas import tpu_sc as plsc

_N = 10000
_E = 320000
_D = 128
_MSG = 64
_R = 32
_NG = 64
_OUT = 10
_CUTOFF = 5.0

_NC = 2
_NS = 16
_NW = _NC * _NS
_EW = _E // _NW

_C2 = 2000
_NCH2 = _EW // _C2


def _d2_body(px_h, py_h, pz_h, src_h, dst_h, out_h, px, py, pz, sv, dv, ov):
    c = lax.axis_index("c")
    s = lax.axis_index("s")
    wid = s * _NC + c
    pltpu.sync_copy(px_h, px)
    pltpu.sync_copy(py_h, py)
    pltpu.sync_copy(pz_h, pz)

    @pl.loop(0, _NCH2)
    def _chunk(j):
        base = wid * _EW + j * _C2
        pltpu.sync_copy(src_h.at[pl.ds(base, _C2)], sv)
        pltpu.sync_copy(dst_h.at[pl.ds(base, _C2)], dv)

        @pl.loop(0, _C2 // 16)
        def _grp(g):
            si = sv[pl.ds(g * 16, 16)]
            di = dv[pl.ds(g * 16, 16)]
            dx = plsc.load_gather(px, [si]) - plsc.load_gather(px, [di])
            dy = plsc.load_gather(py, [si]) - plsc.load_gather(py, [di])
            dz = plsc.load_gather(pz, [si]) - plsc.load_gather(pz, [di])
            ov[pl.ds(g * 16, 16)] = dx * dx + dy * dy + dz * dz

        pltpu.sync_copy(ov, out_h.at[pl.ds(base, _C2)])


@functools.cache
def _build_d2():
    return pl.kernel(
        _d2_body,
        out_type=jax.ShapeDtypeStruct((_E,), jnp.float32),
        mesh=plsc.VectorSubcoreMesh(core_axis_name="c", subcore_axis_name="s",
                                    num_cores=_NC, num_subcores=_NS),
        compiler_params=pltpu.CompilerParams(needs_layout_passes=False,
                                             use_tc_tiling_on_sc=False),
        scratch_types=[
            pltpu.VMEM((_N,), jnp.float32),
            pltpu.VMEM((_N,), jnp.float32),
            pltpu.VMEM((_N,), jnp.float32),
            pltpu.VMEM((_C2,), jnp.int32),
            pltpu.VMEM((_C2,), jnp.int32),
            pltpu.VMEM((_C2,), jnp.float32),
        ],
    )


def _d2_call(*args):
    return _build_d2()(*args)

_CE = 80
_NCHE = _EW // _CE
_NPAIR = (_NCHE - 1) // 2
_NPS = _N // _NS


_NBUF = 4
_KMAIN = _NCHE // _NBUF


def _edge_body(v_h, ef_h, src_h, dst_h, out_h,
               agg, src2, dst2, rows, efb, zb, sg, se, ss):
    c = lax.axis_index("c")
    s = lax.axis_index("s")
    wid = s * _NC + c

    pltpu.sync_copy(src_h.at[wid], src2)
    pltpu.sync_copy(dst_h.at[wid], dst2)

    z16 = jnp.zeros((16,), jnp.float32)

    @pl.loop(0, 125)
    def _zr(i):
        for kk in range(_MSG // 16):
            zb[i, pl.ds(kk * 16, 16)] = z16

    for kk in range(_NPS // 125):
        pltpu.sync_copy(zb, agg.at[pl.ds(s * _NPS + kk * 125, 125)])
    plsc.subcore_barrier()

    def _issue(j, b):
        pltpu.async_copy(
            ef_h.at[pl.ds((wid * _NCHE + j) * (_CE * _MSG // 128),
                          _CE * _MSG // 128)], efb.at[b], se.at[b])
        pltpu.async_copy(v_h.at[src2.at[j]], rows.at[b], sg.at[b])

    def _prefetch(j, b):
        pltpu.make_async_copy(rows.at[b], agg.at[dst2.at[j]], ss.at[b]).wait()
        _issue(j, b)

    def _proc(j, b):
        pltpu.make_async_copy(
            ef_h.at[pl.ds((wid * _NCHE + j) * (_CE * _MSG // 128),
                          _CE * _MSG // 128)], efb.at[b], se.at[b]).wait()
        pltpu.make_async_copy(v_h.at[src2.at[j]], rows.at[b], sg.at[b]).wait()

        @pl.loop(0, _CE)
        def _mul(r):
            for kk in range(_MSG // 16):
                sl = pl.ds(kk * 16, 16)
                sle = pl.ds((r % 2) * _MSG + kk * 16, 16)
                rows[b, r, sl] = rows[b, r, sl] * efb[b, r // 2, sle]

        pltpu.async_copy(rows.at[b], agg.at[dst2.at[j]], ss.at[b], add=True)

    for b in range(_NBUF):
        _issue(b, b)

    @pl.loop(0, _KMAIN)
    def _main(k):
        j0 = _NBUF * k
        _proc(j0 + 0, 0)

        @pl.when(k > 0)
        def _():
            _prefetch(j0 + 3, 3)

        _proc(j0 + 1, 1)
        _prefetch(j0 + 4, 0)
        _proc(j0 + 2, 2)

        @pl.when(k < _KMAIN - 1)
        def _():
            _prefetch(j0 + 5, 1)

        _proc(j0 + 3, 3)

        @pl.when(k < _KMAIN - 1)
        def _():
            _prefetch(j0 + 6, 2)

    _proc(_NBUF * _KMAIN, 0)

    for b in (1, 2, 3, 0):
        j = 120 + b if b else 124
        pltpu.make_async_copy(rows.at[b], agg.at[dst2.at[j]], ss.at[b]).wait()

    plsc.subcore_barrier()
    pltpu.sync_copy(agg.at[pl.ds(s * _NPS, _NPS)],
                    out_h.at[c, pl.ds(s * _NPS, _NPS)])


@functools.cache
def _build_edge():
    return pl.kernel(
        _edge_body,
        out_type=jax.ShapeDtypeStruct((_NC, _N, _MSG), jnp.float32),
        mesh=plsc.VectorSubcoreMesh(core_axis_name="c", subcore_axis_name="s",
                                    num_cores=_NC, num_subcores=_NS),
        compiler_params=pltpu.CompilerParams(needs_layout_passes=False,
                                             use_tc_tiling_on_sc=False),
        scratch_types=[
            pltpu.VMEM_SHARED((_N, _MSG), jnp.float32),
            pltpu.VMEM((_NCHE, _CE), jnp.int32),
            pltpu.VMEM((_NCHE, _CE), jnp.int32),
            pltpu.VMEM((_NBUF, _CE, _MSG), jnp.float32),
            pltpu.VMEM((_NBUF, _CE * _MSG // 128, 128), jnp.float32),
            pltpu.VMEM((125, _MSG), jnp.float32),
            pltpu.SemaphoreType.DMA((_NBUF,)),
            pltpu.SemaphoreType.DMA((_NBUF,)),
            pltpu.SemaphoreType.DMA((_NBUF,)),
        ],
    )


def _edge_call(*args):
    return _build_edge()(*args)


def _embed_body(x_ref, emb_ref, we_ref, be_ref, wv_ref, h0_ref, v0_ref):
    xv = x_ref[...]
    oh = (lax.broadcasted_iota(jnp.int32, (_N, 128), 1) == xv).astype(jnp.float32)
    embw = jnp.dot(emb_ref[...], we_ref[...], preferred_element_type=jnp.float32)
    h0 = jnp.dot(oh, embw, preferred_element_type=jnp.float32) + be_ref[...]
    h0_ref[...] = h0
    v0_ref[...] = jnp.dot(h0, wv_ref[...], preferred_element_type=jnp.float32)


_embed_call = pl.pallas_call(
    _embed_body,
    out_shape=(jax.ShapeDtypeStruct((_N, _D), jnp.float32),
               jax.ShapeDtypeStruct((_N, _MSG), jnp.float32)),
)

_BE = 8000
_GE = _E // _BE

_OFFS = np.linspace(0.0, _CUTOFF, _R).astype(np.float32)
_COEFF = float(-0.5 / (_OFFS[1] - _OFFS[0]) ** 2)


def _ef_body(d2_ref, wr0_ref, wr1_ref, ef0_ref, ef1_ref):
    d2v = d2_ref[...].reshape(2, _BE // 2)
    dist = jnp.sqrt(d2v + 1e-12)
    offs = lax.broadcasted_iota(jnp.int32, (_R, 1), 0).astype(jnp.float32) \
        * (_CUTOFF / (_R - 1))

    def half(p, wr):
        rbf_t = jnp.exp(_COEFF * (dist[p:p + 1] - offs) ** 2)
        rbf = lax.transpose(rbf_t, (1, 0))
        z = jnp.dot(rbf, wr, preferred_element_type=jnp.float32)
        return z * jax.nn.sigmoid(z)

    w0 = wr0_ref[...]
    ef0_ref[...] = jnp.concatenate([half(0, w0), half(1, w0)], axis=1)
    w1 = wr1_ref[...]
    ef1_ref[...] = jnp.concatenate([half(0, w1), half(1, w1)], axis=1)


_ef_call = pl.pallas_call(
    _ef_body,
    grid=(_GE,),
    in_specs=[pl.BlockSpec((1, 2, _BE // 2), lambda i: (i, 0, 0)),
              pl.BlockSpec((_R, _MSG), lambda i: (0, 0)),
              pl.BlockSpec((_R, _MSG), lambda i: (0, 0))],
    out_specs=[pl.BlockSpec((_BE // 2, 128), lambda i: (i, 0)),
               pl.BlockSpec((_BE // 2, 128), lambda i: (i, 0))],
    out_shape=(jax.ShapeDtypeStruct((_E * _MSG // 128, 128), jnp.float32),
               jax.ShapeDtypeStruct((_E * _MSG // 128, 128), jnp.float32)),
)


def _upd_body(h_ref, a0_ref, a1_ref, wo_ref, w1_ref, b1_ref, w2_ref, b2_ref,
              wv_ref, h1_ref, v1_ref):
    agg = a0_ref[...] + a1_ref[...]
    h = h_ref[...] + jnp.dot(agg, wo_ref[...], preferred_element_type=jnp.float32)
    t = jax.nn.gelu(jnp.dot(h, w1_ref[...], preferred_element_type=jnp.float32)
                    + b1_ref[...])
    h = h + jnp.dot(t, w2_ref[...], preferred_element_type=jnp.float32) + b2_ref[...]
    h1_ref[...] = h
    v1_ref[...] = jnp.dot(h, wv_ref[...], preferred_element_type=jnp.float32)


_upd_call = pl.pallas_call(
    _upd_body,
    out_shape=(jax.ShapeDtypeStruct((_N, _D), jnp.float32),
               jax.ShapeDtypeStruct((_N, _MSG), jnp.float32)),
)


def _fin_body(h_ref, a0_ref, a1_ref, wo_ref, w1_ref, b1_ref, w2_ref, b2_ref,
              bat_ref, wf1_ref, bf1_ref, wf2_ref, bf2_ref, wout_ref, bout_ref,
              y_ref):
    agg = a0_ref[...] + a1_ref[...]
    h = h_ref[...] + jnp.dot(agg, wo_ref[...], preferred_element_type=jnp.float32)
    t = jax.nn.gelu(jnp.dot(h, w1_ref[...], preferred_element_type=jnp.float32)
                    + b1_ref[...])
    h = h + jnp.dot(t, w2_ref[...], preferred_element_type=jnp.float32) + b2_ref[...]

    oh = (lax.broadcasted_iota(jnp.int32, (_N, _NG), 1) == bat_ref[...]).astype(jnp.float32)
    sums = lax.dot_general(oh, h, (((0,), (0,)), ((), ())),
                           preferred_element_type=jnp.float32)
    cnt8 = lax.dot_general(oh, jnp.ones((_N, 8), jnp.float32),
                           (((0,), (0,)), ((), ())),
                           preferred_element_type=jnp.float32)
    g = sums / jnp.maximum(cnt8[:, 0:1], 1.0)
    y = jax.nn.gelu(jnp.dot(g, wf1_ref[...], preferred_element_type=jnp.float32)
                    + bf1_ref[...])
    y = jax.nn.gelu(jnp.dot(y, wf2_ref[...], preferred_element_type=jnp.float32)
                    + bf2_ref[...])
    y_ref[...] = jnp.dot(y, wout_ref[...], preferred_element_type=jnp.float32) \
        + bout_ref[...]


_fin_call = pl.pallas_call(
    _fin_body,
    out_shape=jax.ShapeDtypeStruct((_NG, _OUT), jnp.float32),
)


def kernel(x, pos, batch, edge_index, atom_emb, W_embed, b_embed,
           Wv0, Wrbf0, Wo0, fcW1_0, fcb1_0, fcW2_0, fcb2_0,
           Wv1, Wrbf1, Wo1, fcW1_1, fcb1_1, fcW2_1, fcb2_1,
           Wf1, bf1, Wf2, bf2, Wout, bout):
    xi = x.astype(jnp.int32).reshape(_N, 1)
    src = edge_index[0].astype(jnp.int32)
    dst = edge_index[1].astype(jnp.int32)
    px = pos[:, 0]
    py = pos[:, 1]
    pz = pos[:, 2]

    d2 = _d2_call(px, py, pz, src, dst)
    d2i = d2.reshape(_GE, _BE // 2, 2).transpose(0, 2, 1)
    ef0, ef1 = _ef_call(d2i, Wrbf0, Wrbf1)

    emb_pad = jnp.pad(atom_emb, ((0, 128 - atom_emb.shape[0]), (0, 0)))
    h0, v0 = _embed_call(xi, emb_pad, W_embed, b_embed.reshape(1, _D), Wv0)

    src_r = src.reshape(_NW, _NCHE, _CE)
    dst_r = dst.reshape(_NW, _NCHE, _CE)

    aggp0 = _edge_call(v0, ef0, src_r, dst_r)
    h1, v1 = _upd_call(h0, aggp0[0], aggp0[1], Wo0,
                       fcW1_0, fcb1_0.reshape(1, _D),
                       fcW2_0, fcb2_0.reshape(1, _D), Wv1)

    aggp1 = _edge_call(v1, ef1, src_r, dst_r)
    y = _fin_call(h1, aggp1[0], aggp1[1], Wo1,
                  fcW1_1, fcb1_1.reshape(1, _D),
                  fcW2_1, fcb2_1.reshape(1, _D),
                  batch.astype(jnp.int32).reshape(_N, 1),
                  Wf1, bf1.reshape(1, _D), Wf2, bf2.reshape(1, _D),
                  Wout, bout.reshape(1, _OUT))
    return y


# speedup vs baseline: 1.2376x; 1.2376x over previous
import functools

import numpy as np
import jax
import jax.numpy as jnp
from jax import lax
from jax.experimental import pallas as pl
from jax.experimental.pallas import tpu as pltpu
from jax.experimental.pallas import tpu_sc as plsc

_N = 10000
_E = 320000
_D = 128
_MSG = 64
_R = 32
_NG = 64
_OUT = 10
_CUTOFF = 5.0

_NC = 2
_NS = 16
_NW = _NC * _NS
_EW = _E // _NW

_C2 = 2000
_NCH2 = _EW // _C2


def _d2_body(px_h, py_h, pz_h, src_h, dst_h, out_h, px, py, pz, sv, dv, ov):
    c = lax.axis_index("c")
    s = lax.axis_index("s")
    wid = s * _NC + c
    pltpu.sync_copy(px_h, px)
    pltpu.sync_copy(py_h, py)
    pltpu.sync_copy(pz_h, pz)

    @pl.loop(0, _NCH2)
    def _chunk(j):
        base = wid * _EW + j * _C2
        pltpu.sync_copy(src_h.at[pl.ds(base, _C2)], sv)
        pltpu.sync_copy(dst_h.at[pl.ds(base, _C2)], dv)

        @pl.loop(0, _C2 // 16)
        def _grp(g):
            si = sv[pl.ds(g * 16, 16)]
            di = dv[pl.ds(g * 16, 16)]
            dx = plsc.load_gather(px, [si]) - plsc.load_gather(px, [di])
            dy = plsc.load_gather(py, [si]) - plsc.load_gather(py, [di])
            dz = plsc.load_gather(pz, [si]) - plsc.load_gather(pz, [di])
            ov[pl.ds(g * 16, 16)] = dx * dx + dy * dy + dz * dz

        pltpu.sync_copy(ov, out_h.at[pl.ds(base, _C2)])


@functools.cache
def _build_d2():
    return pl.kernel(
        _d2_body,
        out_type=jax.ShapeDtypeStruct((_E,), jnp.float32),
        mesh=plsc.VectorSubcoreMesh(core_axis_name="c", subcore_axis_name="s",
                                    num_cores=_NC, num_subcores=_NS),
        compiler_params=pltpu.CompilerParams(needs_layout_passes=False,
                                             use_tc_tiling_on_sc=False),
        scratch_types=[
            pltpu.VMEM((_N,), jnp.float32),
            pltpu.VMEM((_N,), jnp.float32),
            pltpu.VMEM((_N,), jnp.float32),
            pltpu.VMEM((_C2,), jnp.int32),
            pltpu.VMEM((_C2,), jnp.int32),
            pltpu.VMEM((_C2,), jnp.float32),
        ],
    )


def _d2_call(*args):
    return _build_d2()(*args)

_CE = 80
_NCHE = _EW // _CE
_NPAIR = (_NCHE - 1) // 2
_NPS = _N // _NS


_NBUF = 4
_KMAIN = _NCHE // _NBUF


def _edge_body(v_h, ef_h, src_h, dst_h, out_h,
               agg, src2, dst2, rows, efb, zb, sg, se, ss):
    c = lax.axis_index("c")
    s = lax.axis_index("s")
    wid = s * _NC + c

    pltpu.sync_copy(src_h.at[wid], src2)
    pltpu.sync_copy(dst_h.at[wid], dst2)

    z16 = jnp.zeros((16,), jnp.float32)

    @pl.loop(0, 125)
    def _zr(i):
        for kk in range(_MSG // 16):
            zb[i, pl.ds(kk * 16, 16)] = z16

    for kk in range(_NPS // 125):
        pltpu.sync_copy(zb, agg.at[pl.ds(s * _NPS + kk * 125, 125)])
    plsc.subcore_barrier()

    def _issue(j, b):
        pltpu.async_copy(ef_h.at[pl.ds((wid * _NCHE + j) * _CE, _CE)],
                         efb.at[b], se.at[b])
        pltpu.async_copy(v_h.at[src2.at[j]], rows.at[b], sg.at[b])

    def _prefetch(j, b):
        pltpu.make_async_copy(rows.at[b], agg.at[dst2.at[j]], ss.at[b]).wait()
        _issue(j, b)

    def _proc(j, b):
        pltpu.make_async_copy(ef_h.at[pl.ds((wid * _NCHE + j) * _CE, _CE)],
                              efb.at[b], se.at[b]).wait()
        pltpu.make_async_copy(v_h.at[src2.at[j]], rows.at[b], sg.at[b]).wait()

        @pl.loop(0, _CE)
        def _mul(r):
            for kk in range(_MSG // 16):
                sl = pl.ds(kk * 16, 16)
                rows[b, r, sl] = rows[b, r, sl] * efb[b, r, sl]

        pltpu.async_copy(rows.at[b], agg.at[dst2.at[j]], ss.at[b], add=True)

    for b in range(_NBUF):
        _issue(b, b)

    @pl.loop(0, _KMAIN)
    def _main(k):
        j0 = _NBUF * k
        _proc(j0 + 0, 0)

        @pl.when(k > 0)
        def _():
            _prefetch(j0 + 3, 3)

        _proc(j0 + 1, 1)
        _prefetch(j0 + 4, 0)
        _proc(j0 + 2, 2)

        @pl.when(k < _KMAIN - 1)
        def _():
            _prefetch(j0 + 5, 1)

        _proc(j0 + 3, 3)

        @pl.when(k < _KMAIN - 1)
        def _():
            _prefetch(j0 + 6, 2)

    _proc(_NBUF * _KMAIN, 0)

    for b in (1, 2, 3, 0):
        j = 120 + b if b else 124
        pltpu.make_async_copy(rows.at[b], agg.at[dst2.at[j]], ss.at[b]).wait()

    plsc.subcore_barrier()
    pltpu.sync_copy(agg.at[pl.ds(s * _NPS, _NPS)],
                    out_h.at[c, pl.ds(s * _NPS, _NPS)])


@functools.cache
def _build_edge():
    return pl.kernel(
        _edge_body,
        out_type=jax.ShapeDtypeStruct((_NC, _N, _MSG), jnp.float32),
        mesh=plsc.VectorSubcoreMesh(core_axis_name="c", subcore_axis_name="s",
                                    num_cores=_NC, num_subcores=_NS),
        compiler_params=pltpu.CompilerParams(needs_layout_passes=False,
                                             use_tc_tiling_on_sc=False),
        scratch_types=[
            pltpu.VMEM_SHARED((_N, _MSG), jnp.float32),
            pltpu.VMEM((_NCHE, _CE), jnp.int32),
            pltpu.VMEM((_NCHE, _CE), jnp.int32),
            pltpu.VMEM((_NBUF, _CE, _MSG), jnp.float32),
            pltpu.VMEM((_NBUF, _CE, _MSG), jnp.float32),
            pltpu.VMEM((125, _MSG), jnp.float32),
            pltpu.SemaphoreType.DMA((_NBUF,)),
            pltpu.SemaphoreType.DMA((_NBUF,)),
            pltpu.SemaphoreType.DMA((_NBUF,)),
        ],
    )


def _edge_call(*args):
    return _build_edge()(*args)


def _embed_body(x_ref, emb_ref, we_ref, be_ref, wv_ref, h0_ref, v0_ref):
    xv = x_ref[...]
    oh = (lax.broadcasted_iota(jnp.int32, (_N, 128), 1) == xv).astype(jnp.float32)
    embw = jnp.dot(emb_ref[...], we_ref[...], preferred_element_type=jnp.float32)
    h0 = jnp.dot(oh, embw, preferred_element_type=jnp.float32) + be_ref[...]
    h0_ref[...] = h0
    v0_ref[...] = jnp.dot(h0, wv_ref[...], preferred_element_type=jnp.float32)


_embed_call = pl.pallas_call(
    _embed_body,
    out_shape=(jax.ShapeDtypeStruct((_N, _D), jnp.float32),
               jax.ShapeDtypeStruct((_N, _MSG), jnp.float32)),
)

_BE = 8000
_GE = _E // _BE

_OFFS = np.linspace(0.0, _CUTOFF, _R).astype(np.float32)
_COEFF = float(-0.5 / (_OFFS[1] - _OFFS[0]) ** 2)


def _ef_body(d2_ref, wr0_ref, wr1_ref, ef0_ref, ef1_ref):
    d2v = d2_ref[...].reshape(1, _BE)
    dist = jnp.sqrt(d2v + 1e-12)
    offs = lax.broadcasted_iota(jnp.int32, (_R, 1), 0).astype(jnp.float32) \
        * (_CUTOFF / (_R - 1))
    rbf_t = jnp.exp(_COEFF * (dist - offs) ** 2)
    rbf = lax.transpose(rbf_t, (1, 0))
    z0 = jnp.dot(rbf, wr0_ref[...], preferred_element_type=jnp.float32)
    ef0_ref[...] = z0 * jax.nn.sigmoid(z0)
    z1 = jnp.dot(rbf, wr1_ref[...], preferred_element_type=jnp.float32)
    ef1_ref[...] = z1 * jax.nn.sigmoid(z1)


_ef_call = pl.pallas_call(
    _ef_body,
    grid=(_GE,),
    in_specs=[pl.BlockSpec((1, 1, _BE), lambda i: (i, 0, 0)),
              pl.BlockSpec((_R, _MSG), lambda i: (0, 0)),
              pl.BlockSpec((_R, _MSG), lambda i: (0, 0))],
    out_specs=[pl.BlockSpec((_BE, _MSG), lambda i: (i, 0)),
               pl.BlockSpec((_BE, _MSG), lambda i: (i, 0))],
    out_shape=(jax.ShapeDtypeStruct((_E, _MSG), jnp.float32),
               jax.ShapeDtypeStruct((_E, _MSG), jnp.float32)),
)


def _upd_body(h_ref, a0_ref, a1_ref, wo_ref, w1_ref, b1_ref, w2_ref, b2_ref,
              wv_ref, h1_ref, v1_ref):
    agg = a0_ref[...] + a1_ref[...]
    h = h_ref[...] + jnp.dot(agg, wo_ref[...], preferred_element_type=jnp.float32)
    t = jax.nn.gelu(jnp.dot(h, w1_ref[...], preferred_element_type=jnp.float32)
                    + b1_ref[...])
    h = h + jnp.dot(t, w2_ref[...], preferred_element_type=jnp.float32) + b2_ref[...]
    h1_ref[...] = h
    v1_ref[...] = jnp.dot(h, wv_ref[...], preferred_element_type=jnp.float32)


_upd_call = pl.pallas_call(
    _upd_body,
    out_shape=(jax.ShapeDtypeStruct((_N, _D), jnp.float32),
               jax.ShapeDtypeStruct((_N, _MSG), jnp.float32)),
)


def _fin_body(h_ref, a0_ref, a1_ref, wo_ref, w1_ref, b1_ref, w2_ref, b2_ref,
              bat_ref, wf1_ref, bf1_ref, wf2_ref, bf2_ref, wout_ref, bout_ref,
              y_ref):
    agg = a0_ref[...] + a1_ref[...]
    h = h_ref[...] + jnp.dot(agg, wo_ref[...], preferred_element_type=jnp.float32)
    t = jax.nn.gelu(jnp.dot(h, w1_ref[...], preferred_element_type=jnp.float32)
                    + b1_ref[...])
    h = h + jnp.dot(t, w2_ref[...], preferred_element_type=jnp.float32) + b2_ref[...]

    oh = (lax.broadcasted_iota(jnp.int32, (_N, _NG), 1) == bat_ref[...]).astype(jnp.float32)
    sums = lax.dot_general(oh, h, (((0,), (0,)), ((), ())),
                           preferred_element_type=jnp.float32)
    cnt8 = lax.dot_general(oh, jnp.ones((_N, 8), jnp.float32),
                           (((0,), (0,)), ((), ())),
                           preferred_element_type=jnp.float32)
    g = sums / jnp.maximum(cnt8[:, 0:1], 1.0)
    y = jax.nn.gelu(jnp.dot(g, wf1_ref[...], preferred_element_type=jnp.float32)
                    + bf1_ref[...])
    y = jax.nn.gelu(jnp.dot(y, wf2_ref[...], preferred_element_type=jnp.float32)
                    + bf2_ref[...])
    y_ref[...] = jnp.dot(y, wout_ref[...], preferred_element_type=jnp.float32) \
        + bout_ref[...]


_fin_call = pl.pallas_call(
    _fin_body,
    out_shape=jax.ShapeDtypeStruct((_NG, _OUT), jnp.float32),
)


def kernel(x, pos, batch, edge_index, atom_emb, W_embed, b_embed,
           Wv0, Wrbf0, Wo0, fcW1_0, fcb1_0, fcW2_0, fcb2_0,
           Wv1, Wrbf1, Wo1, fcW1_1, fcb1_1, fcW2_1, fcb2_1,
           Wf1, bf1, Wf2, bf2, Wout, bout):
    xi = x.astype(jnp.int32).reshape(_N, 1)
    src = edge_index[0].astype(jnp.int32)
    dst = edge_index[1].astype(jnp.int32)
    px = pos[:, 0]
    py = pos[:, 1]
    pz = pos[:, 2]

    d2 = _d2_call(px, py, pz, src, dst)
    ef0, ef1 = _ef_call(d2.reshape(_GE, 1, _BE), Wrbf0, Wrbf1)

    emb_pad = jnp.pad(atom_emb, ((0, 128 - atom_emb.shape[0]), (0, 0)))
    h0, v0 = _embed_call(xi, emb_pad, W_embed, b_embed.reshape(1, _D), Wv0)

    src_r = src.reshape(_NW, _NCHE, _CE)
    dst_r = dst.reshape(_NW, _NCHE, _CE)

    aggp0 = _edge_call(v0, ef0, src_r, dst_r)
    h1, v1 = _upd_call(h0, aggp0[0], aggp0[1], Wo0,
                       fcW1_0, fcb1_0.reshape(1, _D),
                       fcW2_0, fcb2_0.reshape(1, _D), Wv1)

    aggp1 = _edge_call(v1, ef1, src_r, dst_r)
    y = _fin_call(h1, aggp1[0], aggp1[1], Wo1,
                  fcW1_1, fcb1_1.reshape(1, _D),
                  fcW2_1, fcb2_1.reshape(1, _D),
                  batch.astype(jnp.int32).reshape(_N, 1),
                  Wf1, bf1.reshape(1, _D), Wf2, bf2.reshape(1, _D),
                  Wout, bout.reshape(1, _OUT))
    return y
